# Initial kernel scaffold; baseline (speedup 1.0000x reference)
import jax
import jax.numpy as jnp
from jax.experimental import pallas as pl


def kernel(x, W1, b1, g1, be1, W2, b2, g2, be2, R, codebook, W3, b3, g3, be3, W4, b4):
    raise NotImplementedError("write your pallas kernel here")


# speedup vs baseline: 2.7073x; 2.7073x over previous
import functools

import jax
import jax.numpy as jnp
from jax import lax
from jax.experimental import pallas as pl
from jax.experimental.pallas import tpu as pltpu
from jax.experimental.pallas import tpu_sc as plsc

H = 4
K = 256
D_IN = 384
D_HID = 256
D_LAT = 384
HD = D_LAT // H
B = 16384

BB = 512

NC = 2
NS = 16
NW = NC * NS
N_IDX = B * H
BPW = N_IDX // NW
CH = 512
HDP = 128


def _ln(x, g, b):
    mu = jnp.mean(x, axis=-1, keepdims=True)
    var = jnp.var(x, axis=-1, keepdims=True)
    return (x - mu) / jnp.sqrt(var + 1e-5) * g + b


def _enc_body(x_ref, W1_ref, b1_ref, g1_ref, be1_ref, W2_ref, b2_ref,
              g2_ref, be2_ref, R_ref, cb_ref, cb2_ref, z_ref, idx_ref):
    x = x_ref[...]
    h = jax.nn.gelu(_ln(x @ W1_ref[...] + b1_ref[...], g1_ref[...], be1_ref[...]))
    z = _ln(h @ W2_ref[...] + b2_ref[...], g2_ref[...], be2_ref[...])
    z_ref[...] = z
    zr = z @ R_ref[...]
    iota_k = lax.broadcasted_iota(jnp.int32, (BB, K), 1)
    for hh in range(H):
        zh = zr[:, hh * HD:(hh + 1) * HD]
        a = jnp.sum(zh * zh, axis=1, keepdims=True)
        p = lax.dot_general(zh, cb_ref[hh], (((1,), (1,)), ((), ())))
        d = a - 2.0 * p + cb2_ref[hh]
        m = jnp.min(d, axis=1, keepdims=True)
        idx = jnp.min(jnp.where(d == m, iota_k, K), axis=1)
        idx_ref[hh, :] = idx


def _dec_body(q_ref, R_ref, W3_ref, b3_ref, g3_ref, be3_ref, W4_ref, b4_ref,
              zq_ref, rec_ref):
    qp = q_ref[...]
    q = jnp.concatenate([qp[:, hh * HDP:hh * HDP + HD] for hh in range(H)],
                        axis=1)
    zq = lax.dot_general(q, R_ref[...], (((1,), (1,)), ((), ())))
    zq_ref[...] = zq
    h2 = jax.nn.gelu(_ln(zq @ W3_ref[...] + b3_ref[...], g3_ref[...], be3_ref[...]))
    rec_ref[...] = h2 @ W4_ref[...] + b4_ref[...]


def _full(shape):
    return pl.BlockSpec(shape, lambda i: tuple(0 for _ in shape))


def _encoder_call(x, W1, b1, g1, be1, W2, b2, g2, be2, R, codebook, cb2):
    return pl.pallas_call(
        _enc_body,
        grid=(B // BB,),
        in_specs=[
            pl.BlockSpec((BB, D_IN), lambda i: (i, 0)),
            _full((D_IN, D_HID)), _full((D_HID,)), _full((D_HID,)), _full((D_HID,)),
            _full((D_HID, D_LAT)), _full((D_LAT,)), _full((D_LAT,)), _full((D_LAT,)),
            _full((D_LAT, D_LAT)),
            _full((H, K, HD)),
            _full((H, K)),
        ],
        out_specs=[
            pl.BlockSpec((BB, D_LAT), lambda i: (i, 0)),
            pl.BlockSpec((H, BB), lambda i: (0, i)),
        ],
        out_shape=[
            jax.ShapeDtypeStruct((B, D_LAT), jnp.float32),
            jax.ShapeDtypeStruct((H, B), jnp.int32),
        ],
        compiler_params=pltpu.CompilerParams(
            dimension_semantics=("parallel",)),
    )(x, W1, b1, g1, be1, W2, b2, g2, be2, R, codebook, cb2)


def _decoder_call(qcat, R, W3, b3, g3, be3, W4, b4):
    return pl.pallas_call(
        _dec_body,
        grid=(B // BB,),
        in_specs=[
            pl.BlockSpec((BB, H * HDP), lambda i: (i, 0)),
            _full((D_LAT, D_LAT)),
            _full((D_LAT, D_HID)), _full((D_HID,)), _full((D_HID,)), _full((D_HID,)),
            _full((D_HID, D_IN)), _full((D_IN,)),
        ],
        out_specs=[
            pl.BlockSpec((BB, D_LAT), lambda i: (i, 0)),
            pl.BlockSpec((BB, D_IN), lambda i: (i, 0)),
        ],
        out_shape=[
            jax.ShapeDtypeStruct((B, D_LAT), jnp.float32),
            jax.ShapeDtypeStruct((B, D_IN), jnp.float32),
        ],
        compiler_params=pltpu.CompilerParams(
            dimension_semantics=("parallel",)),
    )(qcat, R, W3, b3, g3, be3, W4, b4)


def _sc_gather(table, idx_flat):
    mesh = plsc.VectorSubcoreMesh(core_axis_name="c", subcore_axis_name="s")

    @functools.partial(
        pl.kernel,
        mesh=mesh,
        out_type=jax.ShapeDtypeStruct((N_IDX, HDP), jnp.float32),
        scratch_types=[
            pltpu.VMEM((CH,), jnp.int32),
            pltpu.VMEM((CH, HDP), jnp.float32),
            pltpu.SemaphoreType.DMA,
        ],
    )
    def gather_k(table_hbm, idx_hbm, out_hbm, idx_v, rows_v, sem):
        wid = lax.axis_index("s") * NC + lax.axis_index("c")
        base = wid * BPW
        for c in range(BPW // CH):
            off = base + c * CH
            pltpu.sync_copy(idx_hbm.at[pl.ds(off, CH)], idx_v)
            pltpu.async_copy(table_hbm.at[idx_v], rows_v, sem).wait()
            pltpu.sync_copy(rows_v, out_hbm.at[pl.ds(off, CH)])

    return gather_k(table, idx_flat)


def kernel(x, W1, b1, g1, be1, W2, b2, g2, be2, R, codebook, W3, b3, g3, be3,
           W4, b4):
    cb2 = jnp.sum(codebook * codebook, axis=-1)
    z, idx_hb = _encoder_call(x, W1, b1, g1, be1, W2, b2, g2, be2, R,
                              codebook, cb2)
    indices = idx_hb.T
    idx_flat = (idx_hb + (K * jnp.arange(H, dtype=jnp.int32))[:, None]).T.reshape(-1)
    table = jnp.pad(codebook.reshape(H * K, HD), ((0, 0), (0, HDP - HD)))
    q = _sc_gather(table, idx_flat)
    qcat = q.reshape(B, H * HDP)
    z_q, reconstructed = _decoder_call(qcat, R, W3, b3, g3, be3, W4, b4)
    return (reconstructed, indices, z, z_q)


# speedup vs baseline: 3.6562x; 1.3505x over previous
import functools

import jax
import jax.numpy as jnp
from jax import lax
from jax.experimental import pallas as pl
from jax.experimental.pallas import tpu as pltpu
from jax.experimental.pallas import tpu_sc as plsc

H = 4
K = 256
D_IN = 384
D_HID = 256
D_LAT = 384
HD = D_LAT // H
B = 16384

BB = 512

NC = 2
NS = 16
NW = NC * NS
N_IDX = B * H
BPW = N_IDX // NW
CH = 256
HDP = 128


def _ln(x, g, b):
    mu = jnp.mean(x, axis=-1, keepdims=True)
    var = jnp.var(x, axis=-1, keepdims=True)
    return (x - mu) / jnp.sqrt(var + 1e-5) * g + b


def _enc_body(x_ref, W1_ref, b1_ref, g1_ref, be1_ref, W2_ref, b2_ref,
              g2_ref, be2_ref, R_ref, cb_ref, cb2t_ref, z_ref, idx_ref):
    x = x_ref[...]
    h = jax.nn.gelu(_ln(x @ W1_ref[...] + b1_ref[...], g1_ref[...], be1_ref[...]))
    z = _ln(h @ W2_ref[...] + b2_ref[...], g2_ref[...], be2_ref[...])
    z_ref[...] = z
    zrT = lax.dot_general(R_ref[...], z, (((0,), (1,)), ((), ())))
    iota_k = lax.broadcasted_iota(jnp.int32, (K, BB), 0).astype(jnp.float32)
    for hh in range(H):
        zhT = zrT[hh * HD:(hh + 1) * HD, :]
        aT = jnp.sum(zhT * zhT, axis=0, keepdims=True)
        pT = lax.dot_general(cb_ref[hh], zhT, (((1,), (0,)), ((), ())))
        dT = aT - 2.0 * pT + cb2t_ref[:, hh:hh + 1]
        m = jnp.min(dT, axis=0, keepdims=True)
        idxf = jnp.min(jnp.where(dT == m, iota_k, float(K)), axis=0)
        idx_ref[hh, :] = idxf.astype(jnp.int32)


def _dec_body(q_ref, RTp_ref, W3_ref, b3_ref, g3_ref, be3_ref, W4_ref, b4_ref,
              zq_ref, rec_ref):
    qp = q_ref[...]
    zq = lax.dot_general(qp.astype(jnp.bfloat16), RTp_ref[...],
                         (((1,), (0,)), ((), ())),
                         preferred_element_type=jnp.float32)
    zq_ref[...] = zq
    h2 = jax.nn.gelu(_ln(
        lax.dot_general(zq.astype(jnp.bfloat16), W3_ref[...],
                        (((1,), (0,)), ((), ())),
                        preferred_element_type=jnp.float32) + b3_ref[...],
        g3_ref[...], be3_ref[...]))
    rec_ref[...] = lax.dot_general(h2.astype(jnp.bfloat16), W4_ref[...],
                                   (((1,), (0,)), ((), ())),
                                   preferred_element_type=jnp.float32) + b4_ref[...]


def _full(shape):
    return pl.BlockSpec(shape, lambda i: tuple(0 for _ in shape))


def _encoder_call(x, W1, b1, g1, be1, W2, b2, g2, be2, R, codebook, cb2t):
    return pl.pallas_call(
        _enc_body,
        grid=(B // BB,),
        in_specs=[
            pl.BlockSpec((BB, D_IN), lambda i: (i, 0)),
            _full((D_IN, D_HID)), _full((D_HID,)), _full((D_HID,)), _full((D_HID,)),
            _full((D_HID, D_LAT)), _full((D_LAT,)), _full((D_LAT,)), _full((D_LAT,)),
            _full((D_LAT, D_LAT)),
            _full((H, K, HD)),
            _full((K, H)),
        ],
        out_specs=[
            pl.BlockSpec((BB, D_LAT), lambda i: (i, 0)),
            pl.BlockSpec((H, BB), lambda i: (0, i)),
        ],
        out_shape=[
            jax.ShapeDtypeStruct((B, D_LAT), jnp.float32),
            jax.ShapeDtypeStruct((H, B), jnp.int32),
        ],
        compiler_params=pltpu.CompilerParams(
            dimension_semantics=("parallel",)),
    )(x, W1, b1, g1, be1, W2, b2, g2, be2, R, codebook, cb2t)


def _decoder_call(qcat, RTp, W3, b3, g3, be3, W4, b4):
    return pl.pallas_call(
        _dec_body,
        grid=(B // BB,),
        in_specs=[
            pl.BlockSpec((BB, H * HDP), lambda i: (i, 0)),
            _full((H * HDP, D_LAT)),
            _full((D_LAT, D_HID)), _full((D_HID,)), _full((D_HID,)), _full((D_HID,)),
            _full((D_HID, D_IN)), _full((D_IN,)),
        ],
        out_specs=[
            pl.BlockSpec((BB, D_LAT), lambda i: (i, 0)),
            pl.BlockSpec((BB, D_IN), lambda i: (i, 0)),
        ],
        out_shape=[
            jax.ShapeDtypeStruct((B, D_LAT), jnp.float32),
            jax.ShapeDtypeStruct((B, D_IN), jnp.float32),
        ],
        compiler_params=pltpu.CompilerParams(
            dimension_semantics=("parallel",)),
    )(qcat, RTp, W3, b3, g3, be3, W4, b4)


def _sc_gather(table, idx_flat):
    mesh = plsc.VectorSubcoreMesh(core_axis_name="c", subcore_axis_name="s")

    @functools.partial(
        pl.kernel,
        mesh=mesh,
        out_type=jax.ShapeDtypeStruct((N_IDX, HDP), jnp.float32),
        scratch_types=[
            pltpu.VMEM((CH,), jnp.int32),
            pltpu.VMEM((CH,), jnp.int32),
            pltpu.VMEM((CH, HDP), jnp.float32),
            pltpu.VMEM((CH, HDP), jnp.float32),
            pltpu.SemaphoreType.DMA,
            pltpu.SemaphoreType.DMA,
        ],
    )
    def gather_k(table_hbm, idx_hbm, out_hbm, idx0, idx1, rows0, rows1,
                 sem0, sem1):
        wid = lax.axis_index("s") * NC + lax.axis_index("c")
        base = wid * BPW
        idxb = (idx0, idx1)
        rows = (rows0, rows1)
        sems = (sem0, sem1)
        num = BPW // CH
        cps = [None, None]
        for c in range(num):
            bu = c % 2
            pltpu.sync_copy(idx_hbm.at[pl.ds(base + c * CH, CH)], idxb[bu])
            cps[bu] = pltpu.async_copy(table_hbm.at[idxb[bu]], rows[bu],
                                       sems[bu])
            if c > 0:
                cps[1 - bu].wait()
                pltpu.sync_copy(rows[1 - bu],
                                out_hbm.at[pl.ds(base + (c - 1) * CH, CH)])
        last = (num - 1) % 2
        cps[last].wait()
        pltpu.sync_copy(rows[last], out_hbm.at[pl.ds(base + (num - 1) * CH, CH)])

    return gather_k(table, idx_flat)


def kernel(x, W1, b1, g1, be1, W2, b2, g2, be2, R, codebook, W3, b3, g3, be3,
           W4, b4):
    cb2t = jnp.sum(codebook * codebook, axis=-1).T
    z, idx_hb = _encoder_call(x, W1, b1, g1, be1, W2, b2, g2, be2, R,
                              codebook, cb2t)
    indices = idx_hb.T
    idx_flat = (idx_hb + (K * jnp.arange(H, dtype=jnp.int32))[:, None]).T.reshape(-1)
    table = jnp.pad(codebook.reshape(H * K, HD), ((0, 0), (0, HDP - HD)))
    q = _sc_gather(table, idx_flat)
    qcat = q.reshape(B, H * HDP)
    RTp = jnp.pad(R.T.reshape(H, HD, D_LAT), ((0, 0), (0, HDP - HD), (0, 0))
                  ).reshape(H * HDP, D_LAT).astype(jnp.bfloat16)
    z_q, reconstructed = _decoder_call(qcat, RTp, W3.astype(jnp.bfloat16), b3,
                                       g3, be3, W4.astype(jnp.bfloat16), b4)
    return (reconstructed, indices, z, z_q)
